# baseline (speedup 1.0000x reference)
import functools
import math

import jax
import jax.numpy as jnp
from jax import lax
from jax.experimental import pallas as pl
from jax.experimental.pallas import tpu as pltpu
from jax.experimental.pallas import tpu_sc as plsc

EPS = 0.1
DROP_WORST_RATIO = 0.2

_NC, _NS, _L = 2, 16, 16
_NW = _NC * _NS
_C = 2048
_N_SC = 2048


def _row_stats_body(scores_ref, target_ref, loss_ref, *, on, off, const):
    x = scores_ref[...]
    m = jnp.max(x, axis=1, keepdims=True)
    se = jnp.sum(jnp.exp(x - m), axis=1, keepdims=True)
    lse = m + jnp.log(se)
    rs = jnp.sum(x, axis=1, keepdims=True)
    t = target_ref[...]
    cols = lax.broadcasted_iota(jnp.int32, x.shape, 1)
    st = jnp.sum(jnp.where(cols == t, x, 0.0), axis=1, keepdims=True)
    loss_ref[...] = (const + lse) - off * rs - (on - off) * st


def _sc_stats_body(scores_hbm, tgt_hbm, m_hbm, se_hbm, rs_hbm, st_hbm,
                   tgt_v, om_v, ose_v, ors_v, ost_v, buf0, buf1, buft,
                   sem0, sem1, *, V_al, row_base, rows_w):
    wid = lax.axis_index("s") * _NC + lax.axis_index("c")
    w0 = wid * rows_w
    row0 = row_base + w0
    pltpu.sync_copy(tgt_hbm.at[pl.ds(w0 * _L, rows_w * _L)], tgt_v)
    n_full = V_al // _C
    rem = V_al % _C
    n_chunks = n_full + (1 if rem else 0)
    chunks = [(ci * _C, _C, [buf0, buf1][ci % 2]) for ci in range(n_full)]
    if rem:
        chunks.append((n_full * _C, rem, buft))
    sems = [sem0, sem1]
    lane = lax.iota(jnp.int32, _L)
    neg_inf = jnp.full((_L,), -float("inf"), jnp.float32)
    zeros = jnp.zeros((_L,), jnp.float32)

    def group_body(g, carry):
        gr0 = g * _L
        rows = pl.ds(row0 + gr0, _L)
        for ci in range(n_chunks):
            start, width, buf = chunks[ci]
            pltpu.sync_copy(scores_hbm.at[rows, pl.ds(start, width)], buf)
            nv = width // _L

            def row_body(r, carry2, buf=buf, start=start, first=(ci == 0)):
                slot = pl.multiple_of((gr0 + r) * _L, _L)
                t16 = tgt_v[pl.ds(slot, _L)]
                c0 = (t16 - start) - lane

                def p1(i, c):
                    m, rs, st, cc = c
                    x = buf[r, pl.ds(pl.multiple_of(i * _L, _L), _L)]
                    st = st + jnp.where(cc == 0, x, 0.0)
                    return (jnp.maximum(m, x), rs + x, st, cc - _L)

                m16, rs16, st16, _ = lax.fori_loop(
                    0, nv, p1, (neg_inf, zeros, zeros, c0), unroll=16)

                def p2(i, se):
                    x = buf[r, pl.ds(pl.multiple_of(i * _L, _L), _L)]
                    return se + jnp.exp(x - m16)

                se16 = lax.fori_loop(0, nv, p2, zeros, unroll=16)

                if first:
                    om_v[pl.ds(slot, _L)] = m16
                    ose_v[pl.ds(slot, _L)] = se16
                    ors_v[pl.ds(slot, _L)] = rs16
                    ost_v[pl.ds(slot, _L)] = st16
                else:
                    m_old = om_v[pl.ds(slot, _L)]
                    m_new = jnp.maximum(m_old, m16)
                    om_v[pl.ds(slot, _L)] = m_new
                    ose_v[pl.ds(slot, _L)] = (
                        ose_v[pl.ds(slot, _L)] * jnp.exp(m_old - m_new)
                        + se16 * jnp.exp(m16 - m_new))
                    ors_v[pl.ds(slot, _L)] = ors_v[pl.ds(slot, _L)] + rs16
                    ost_v[pl.ds(slot, _L)] = ost_v[pl.ds(slot, _L)] + st16
                return carry2

            lax.fori_loop(0, _L, row_body, 0)
        return carry

    lax.fori_loop(0, rows_w // _L, group_body, 0)
    pltpu.sync_copy(om_v, m_hbm.at[pl.ds(w0 * _L, rows_w * _L)])
    pltpu.sync_copy(ose_v, se_hbm.at[pl.ds(w0 * _L, rows_w * _L)])
    pltpu.sync_copy(ors_v, rs_hbm.at[pl.ds(w0 * _L, rows_w * _L)])
    pltpu.sync_copy(ost_v, st_hbm.at[pl.ds(w0 * _L, rows_w * _L)])


def _sc_stats(scores, tgt_sc, V_al, row_base, n_sc):
    rows_w = n_sc // _NW
    rem = V_al % _C
    mesh = plsc.VectorSubcoreMesh(core_axis_name="c", subcore_axis_name="s")
    fn = pl.kernel(
        functools.partial(_sc_stats_body, V_al=V_al, row_base=row_base,
                          rows_w=rows_w),
        mesh=mesh,
        out_type=[jax.ShapeDtypeStruct((n_sc * _L,), jnp.float32)] * 4,
        scratch_types=[
            pltpu.VMEM((rows_w * _L,), jnp.int32),
            pltpu.VMEM((rows_w * _L,), jnp.float32),
            pltpu.VMEM((rows_w * _L,), jnp.float32),
            pltpu.VMEM((rows_w * _L,), jnp.float32),
            pltpu.VMEM((rows_w * _L,), jnp.float32),
            pltpu.VMEM((_L, _C), jnp.float32),
            pltpu.VMEM((_L, _C), jnp.float32),
            pltpu.VMEM((_L, max(rem, _L)), jnp.float32),
            pltpu.SemaphoreType.DMA,
            pltpu.SemaphoreType.DMA,
        ],
    )
    return fn(scores, tgt_sc)


def _finish_body(m_ref, se_ref, rs_ref, st_ref, strip_ref, target_ref,
                 loss_ref, *, on, off, const, v_al, w_tail):
    m16 = m_ref[...]
    m_row = jnp.max(m16, axis=1, keepdims=True)
    se_row = jnp.sum(se_ref[...] * jnp.exp(m16 - m_row), axis=1,
                     keepdims=True)
    rs_row = jnp.sum(rs_ref[...], axis=1, keepdims=True)
    st_row = jnp.sum(st_ref[...], axis=1, keepdims=True)

    xsr = strip_ref[...]
    lanec = lax.broadcasted_iota(jnp.int32, xsr.shape, 1)
    valid = lanec < w_tail
    xs = jnp.where(valid, xsr, -jnp.inf)
    mt = jnp.max(xs, axis=1, keepdims=True)
    sett = jnp.sum(jnp.exp(xs - mt), axis=1, keepdims=True)
    rst = jnp.sum(jnp.where(valid, xsr, 0.0), axis=1, keepdims=True)

    m_all = jnp.maximum(m_row, mt)
    se_all = (se_row * jnp.exp(m_row - m_all) + sett * jnp.exp(mt - m_all))
    rs_all = rs_row + rst

    t = target_ref[...]
    cols = v_al + lax.broadcasted_iota(jnp.int32, xs.shape, 1)
    st_tail = jnp.sum(jnp.where(cols == t, xs, 0.0), axis=1, keepdims=True)
    st_all = jnp.where(t >= v_al, st_tail, st_row)

    loss_ref[...] = ((const + m_all + jnp.log(se_all))
                     - off * rs_all - (on - off) * st_all)


def _select_mean_body(loss_a_ref, loss_b_ref, out_ref, *, k):
    lv = jnp.concatenate([loss_a_ref[...], loss_b_ref[...]], axis=0)
    u = lax.bitcast_convert_type(lv, jnp.uint32)
    key = jnp.where(u >= jnp.uint32(0x80000000), ~u, u | jnp.uint32(0x80000000))

    def body(i, prefix):
        bit = jnp.uint32(31) - i.astype(jnp.uint32)
        trial = prefix | (jnp.uint32(1) << bit)
        c = jnp.sum((key < trial).astype(jnp.int32))
        return jnp.where(c < k, trial, prefix)

    kth = lax.fori_loop(0, 32, body, jnp.uint32(0))
    below = key < kth
    cnt = jnp.sum(below.astype(jnp.int32))
    ssum = jnp.sum(jnp.where(below, lv, 0.0))
    kth_bits = jnp.where(kth >= jnp.uint32(0x80000000),
                         kth ^ jnp.uint32(0x80000000), ~kth)
    kth_val = lax.bitcast_convert_type(kth_bits, jnp.float32)
    total = ssum + (k - cnt).astype(jnp.float32) * kth_val
    out_ref[...] = jnp.broadcast_to(total / jnp.float32(k), (1, 1))


def kernel(scores, target):
    N, V = scores.shape
    on = 1.0 - EPS
    off = EPS / (V - 1)
    const = on * math.log(on) + (V - 1) * (off * math.log(off))
    k = int(N * (1.0 - DROP_WORST_RATIO))
    target = target.astype(jnp.int32)

    n_sc = _N_SC
    n_tc = N - n_sc
    V_al = (V // 128) * 128
    W = V - V_al

    tgt_splat = jnp.broadcast_to(target[n_tc:, None], (n_sc, _L))
    m_sc, se_sc, rs_sc, st_sc = _sc_stats(scores, tgt_splat.reshape(-1),
                                          V_al, n_tc, n_sc)

    R = 64
    loss_tc = pl.pallas_call(
        functools.partial(_row_stats_body, on=on, off=off, const=const),
        grid=(n_tc // R,),
        in_specs=[
            pl.BlockSpec((R, V), lambda i: (i, 0)),
            pl.BlockSpec((R, 1), lambda i: (i, 0)),
        ],
        out_specs=pl.BlockSpec((R, 1), lambda i: (i, 0)),
        out_shape=jax.ShapeDtypeStruct((n_tc, 1), jnp.float32),
    )(scores, target[:n_tc].reshape(n_tc, 1))

    Rt = 512
    tb = n_tc // Rt
    cb = V_al // 128
    loss_sc = pl.pallas_call(
        functools.partial(_finish_body, on=on, off=off, const=const,
                          v_al=V_al, w_tail=W),
        grid=(n_sc // Rt,),
        in_specs=[
            pl.BlockSpec((Rt, _L), lambda i: (i, 0)),
            pl.BlockSpec((Rt, _L), lambda i: (i, 0)),
            pl.BlockSpec((Rt, _L), lambda i: (i, 0)),
            pl.BlockSpec((Rt, _L), lambda i: (i, 0)),
            pl.BlockSpec((Rt, 128), lambda i: (i + tb, cb)),
            pl.BlockSpec((Rt, 1), lambda i: (i, 0)),
        ],
        out_specs=pl.BlockSpec((Rt, 1), lambda i: (i, 0)),
        out_shape=jax.ShapeDtypeStruct((n_sc, 1), jnp.float32),
    )(m_sc.reshape(n_sc, _L), se_sc.reshape(n_sc, _L),
      rs_sc.reshape(n_sc, _L), st_sc.reshape(n_sc, _L),
      scores, target[n_tc:].reshape(n_sc, 1))

    out = pl.pallas_call(
        functools.partial(_select_mean_body, k=k),
        out_shape=jax.ShapeDtypeStruct((1, 1), jnp.float32),
    )(loss_tc.reshape(n_tc // 128, 128),
      loss_sc.reshape(n_sc // 128, 128))
    return out.reshape(())


# speedup vs baseline: 1.0746x; 1.0746x over previous
import functools
import math

import jax
import jax.numpy as jnp
from jax import lax
from jax.experimental import pallas as pl
from jax.experimental.pallas import tpu as pltpu
from jax.experimental.pallas import tpu_sc as plsc

EPS = 0.1
DROP_WORST_RATIO = 0.2

_NC, _NS, _L = 2, 16, 16
_NW = _NC * _NS
_C = 2048
_N_SC = 2048


def _row_stats_body(scores_ref, target_ref, loss_ref, *, on, off, const):
    x = scores_ref[...]
    m = jnp.max(x, axis=1, keepdims=True)
    se = jnp.sum(jnp.exp(x - m), axis=1, keepdims=True)
    lse = m + jnp.log(se)
    rs = jnp.sum(x, axis=1, keepdims=True)
    t = target_ref[...]
    cols = lax.broadcasted_iota(jnp.int32, x.shape, 1)
    st = jnp.sum(jnp.where(cols == t, x, 0.0), axis=1, keepdims=True)
    loss_ref[...] = (const + lse) - off * rs - (on - off) * st


def _sc_stats_body(scores_hbm, tgt_hbm, m_hbm, se_hbm, rs_hbm, st_hbm,
                   tgt_v, om_v, ose_v, ors_v, ost_v, buf0, buf1, buft,
                   sem0, sem1, *, V_al, row_base, rows_w):
    wid = lax.axis_index("s") * _NC + lax.axis_index("c")
    w0 = wid * rows_w
    row0 = row_base + w0
    pltpu.sync_copy(tgt_hbm.at[pl.ds(w0 * _L, rows_w * _L)], tgt_v)
    n_full = V_al // _C
    rem = V_al % _C
    n_chunks = n_full + (1 if rem else 0)
    chunks = [(ci * _C, _C, [buf0, buf1][ci % 2]) for ci in range(n_full)]
    if rem:
        chunks.append((n_full * _C, rem, buft))
    sems = [sem0, sem1]
    lane = lax.iota(jnp.int32, _L)
    neg_inf = jnp.full((_L,), -float("inf"), jnp.float32)
    zeros = jnp.zeros((_L,), jnp.float32)

    def start_chunk(rows, ci):
        start, width, buf = chunks[ci]
        sem = sems[ci % 2]
        return [
            pltpu.async_copy(
                scores_hbm.at[rows, pl.ds(start + jj * 128, 128)],
                buf.at[:, pl.ds(jj * 128, 128)], sem)
            for jj in range(width // 128)
        ]

    def group_body(g, carry):
        gr0 = g * _L
        rows = pl.ds(row0 + gr0, _L)
        cps = [None] * n_chunks
        cps[0] = start_chunk(rows, 0)
        for ci in range(n_chunks):
            start, width, buf = chunks[ci]
            if ci + 1 < n_chunks:
                cps[ci + 1] = start_chunk(rows, ci + 1)
            for c in cps[ci]:
                c.wait()
            nv = width // _L

            def row_body(r, carry2, buf=buf, start=start, first=(ci == 0)):
                slot = pl.multiple_of((gr0 + r) * _L, _L)
                t16 = tgt_v[pl.ds(slot, _L)]
                c0 = (t16 - start) - lane

                def p1(i, c):
                    m, rs, st, cc = c
                    x = buf[r, pl.ds(pl.multiple_of(i * _L, _L), _L)]
                    st = st + jnp.where(cc == 0, x, 0.0)
                    return (jnp.maximum(m, x), rs + x, st, cc - _L)

                m16, rs16, st16, _ = lax.fori_loop(
                    0, nv, p1, (neg_inf, zeros, zeros, c0), unroll=16)

                def p2(i, se):
                    x = buf[r, pl.ds(pl.multiple_of(i * _L, _L), _L)]
                    return se + jnp.exp(x - m16)

                se16 = lax.fori_loop(0, nv, p2, zeros, unroll=16)

                if first:
                    om_v[pl.ds(slot, _L)] = m16
                    ose_v[pl.ds(slot, _L)] = se16
                    ors_v[pl.ds(slot, _L)] = rs16
                    ost_v[pl.ds(slot, _L)] = st16
                else:
                    m_old = om_v[pl.ds(slot, _L)]
                    m_new = jnp.maximum(m_old, m16)
                    om_v[pl.ds(slot, _L)] = m_new
                    ose_v[pl.ds(slot, _L)] = (
                        ose_v[pl.ds(slot, _L)] * jnp.exp(m_old - m_new)
                        + se16 * jnp.exp(m16 - m_new))
                    ors_v[pl.ds(slot, _L)] = ors_v[pl.ds(slot, _L)] + rs16
                    ost_v[pl.ds(slot, _L)] = ost_v[pl.ds(slot, _L)] + st16
                return carry2

            lax.fori_loop(0, _L, row_body, 0)
        return carry

    lax.fori_loop(0, rows_w // _L, group_body, 0)
    pltpu.sync_copy(om_v, m_hbm.at[pl.ds(w0 * _L, rows_w * _L)])
    pltpu.sync_copy(ose_v, se_hbm.at[pl.ds(w0 * _L, rows_w * _L)])
    pltpu.sync_copy(ors_v, rs_hbm.at[pl.ds(w0 * _L, rows_w * _L)])
    pltpu.sync_copy(ost_v, st_hbm.at[pl.ds(w0 * _L, rows_w * _L)])


def _sc_stats(scores, tgt_sc, V_al, row_base, n_sc):
    rows_w = n_sc // _NW
    rem = V_al % _C
    mesh = plsc.VectorSubcoreMesh(core_axis_name="c", subcore_axis_name="s")
    fn = pl.kernel(
        functools.partial(_sc_stats_body, V_al=V_al, row_base=row_base,
                          rows_w=rows_w),
        mesh=mesh,
        out_type=[jax.ShapeDtypeStruct((n_sc * _L,), jnp.float32)] * 4,
        scratch_types=[
            pltpu.VMEM((rows_w * _L,), jnp.int32),
            pltpu.VMEM((rows_w * _L,), jnp.float32),
            pltpu.VMEM((rows_w * _L,), jnp.float32),
            pltpu.VMEM((rows_w * _L,), jnp.float32),
            pltpu.VMEM((rows_w * _L,), jnp.float32),
            pltpu.VMEM((_L, _C), jnp.float32),
            pltpu.VMEM((_L, _C), jnp.float32),
            pltpu.VMEM((_L, max(rem, _L)), jnp.float32),
            pltpu.SemaphoreType.DMA,
            pltpu.SemaphoreType.DMA,
        ],
    )
    return fn(scores, tgt_sc)


def _finish_body(m_ref, se_ref, rs_ref, st_ref, strip_ref, target_ref,
                 loss_ref, *, on, off, const, v_al, w_tail):
    m16 = m_ref[...]
    m_row = jnp.max(m16, axis=1, keepdims=True)
    se_row = jnp.sum(se_ref[...] * jnp.exp(m16 - m_row), axis=1,
                     keepdims=True)
    rs_row = jnp.sum(rs_ref[...], axis=1, keepdims=True)
    st_row = jnp.sum(st_ref[...], axis=1, keepdims=True)

    xsr = strip_ref[...]
    lanec = lax.broadcasted_iota(jnp.int32, xsr.shape, 1)
    valid = lanec < w_tail
    xs = jnp.where(valid, xsr, -jnp.inf)
    mt = jnp.max(xs, axis=1, keepdims=True)
    sett = jnp.sum(jnp.exp(xs - mt), axis=1, keepdims=True)
    rst = jnp.sum(jnp.where(valid, xsr, 0.0), axis=1, keepdims=True)

    m_all = jnp.maximum(m_row, mt)
    se_all = (se_row * jnp.exp(m_row - m_all) + sett * jnp.exp(mt - m_all))
    rs_all = rs_row + rst

    t = target_ref[...]
    cols = v_al + lax.broadcasted_iota(jnp.int32, xs.shape, 1)
    st_tail = jnp.sum(jnp.where(cols == t, xs, 0.0), axis=1, keepdims=True)
    st_all = jnp.where(t >= v_al, st_tail, st_row)

    loss_ref[...] = ((const + m_all + jnp.log(se_all))
                     - off * rs_all - (on - off) * st_all)


def _select_mean_body(loss_a_ref, loss_b_ref, out_ref, *, k):
    lv = jnp.concatenate([loss_a_ref[...], loss_b_ref[...]], axis=0)
    u = lax.bitcast_convert_type(lv, jnp.uint32)
    key = jnp.where(u >= jnp.uint32(0x80000000), ~u, u | jnp.uint32(0x80000000))

    def body(i, prefix):
        bit = jnp.uint32(31) - i.astype(jnp.uint32)
        trial = prefix | (jnp.uint32(1) << bit)
        c = jnp.sum((key < trial).astype(jnp.int32))
        return jnp.where(c < k, trial, prefix)

    kth = lax.fori_loop(0, 32, body, jnp.uint32(0))
    below = key < kth
    cnt = jnp.sum(below.astype(jnp.int32))
    ssum = jnp.sum(jnp.where(below, lv, 0.0))
    kth_bits = jnp.where(kth >= jnp.uint32(0x80000000),
                         kth ^ jnp.uint32(0x80000000), ~kth)
    kth_val = lax.bitcast_convert_type(kth_bits, jnp.float32)
    total = ssum + (k - cnt).astype(jnp.float32) * kth_val
    out_ref[...] = jnp.broadcast_to(total / jnp.float32(k), (1, 1))


def kernel(scores, target):
    N, V = scores.shape
    on = 1.0 - EPS
    off = EPS / (V - 1)
    const = on * math.log(on) + (V - 1) * (off * math.log(off))
    k = int(N * (1.0 - DROP_WORST_RATIO))
    target = target.astype(jnp.int32)

    n_sc = _N_SC
    n_tc = N - n_sc
    V_al = (V // 128) * 128
    W = V - V_al

    tgt_splat = jnp.broadcast_to(target[n_tc:, None], (n_sc, _L))
    m_sc, se_sc, rs_sc, st_sc = _sc_stats(scores, tgt_splat.reshape(-1),
                                          V_al, n_tc, n_sc)

    R = 64
    loss_tc = pl.pallas_call(
        functools.partial(_row_stats_body, on=on, off=off, const=const),
        grid=(n_tc // R,),
        in_specs=[
            pl.BlockSpec((R, V), lambda i: (i, 0)),
            pl.BlockSpec((R, 1), lambda i: (i, 0)),
        ],
        out_specs=pl.BlockSpec((R, 1), lambda i: (i, 0)),
        out_shape=jax.ShapeDtypeStruct((n_tc, 1), jnp.float32),
    )(scores, target[:n_tc].reshape(n_tc, 1))

    Rt = 512
    tb = n_tc // Rt
    cb = V_al // 128
    loss_sc = pl.pallas_call(
        functools.partial(_finish_body, on=on, off=off, const=const,
                          v_al=V_al, w_tail=W),
        grid=(n_sc // Rt,),
        in_specs=[
            pl.BlockSpec((Rt, _L), lambda i: (i, 0)),
            pl.BlockSpec((Rt, _L), lambda i: (i, 0)),
            pl.BlockSpec((Rt, _L), lambda i: (i, 0)),
            pl.BlockSpec((Rt, _L), lambda i: (i, 0)),
            pl.BlockSpec((Rt, 128), lambda i: (i + tb, cb)),
            pl.BlockSpec((Rt, 1), lambda i: (i, 0)),
        ],
        out_specs=pl.BlockSpec((Rt, 1), lambda i: (i, 0)),
        out_shape=jax.ShapeDtypeStruct((n_sc, 1), jnp.float32),
    )(m_sc.reshape(n_sc, _L), se_sc.reshape(n_sc, _L),
      rs_sc.reshape(n_sc, _L), st_sc.reshape(n_sc, _L),
      scores, target[n_tc:].reshape(n_sc, 1))

    out = pl.pallas_call(
        functools.partial(_select_mean_body, k=k),
        out_shape=jax.ShapeDtypeStruct((1, 1), jnp.float32),
    )(loss_tc.reshape(n_tc // 128, 128),
      loss_sc.reshape(n_sc // 128, 128))
    return out.reshape(())
